# baseline (speedup 1.0000x reference)
import functools

import jax
import jax.numpy as jnp
from jax import lax
from jax.experimental import pallas as pl
from jax.experimental.pallas import tpu as pltpu
from jax.experimental.pallas import tpu_sc as plsc

N = 10000
E = 320000
D_IN = 128
D_H = 128
D_OUT = 64

NC = 2
NS = 16
NT = NC * NS
K = 40
EPT = E // NT
NCHUNK = EPT // K
NP = 10016
RPT = NP // NS
HL = 16
NB = 5
ROUNDS = NCHUNK // NB
KH = 80
NCHUNK_H = EPT // KH


def _hist_body(srcr_hbm, dstr_hbm, out_hbm, src_v, dst_v, ones_v, zero_v,
               ha_sh, hb_sh, sem):
    c = lax.axis_index("c")
    s = lax.axis_index("s")

    pltpu.sync_copy(srcr_hbm.at[c, s], src_v)
    pltpu.sync_copy(dstr_hbm.at[c, s], dst_v)

    def fill_ones(i, carry):
        ones_v[i, :] = jnp.ones((HL,), jnp.float32)
        return carry

    lax.fori_loop(0, KH, fill_ones, 0)

    def fill_zero(i, carry):
        zero_v[i, :] = jnp.zeros((HL,), jnp.float32)
        return carry

    lax.fori_loop(0, RPT, fill_zero, 0)

    pltpu.sync_copy(zero_v, ha_sh.at[pl.ds(s * RPT, RPT)])
    pltpu.sync_copy(zero_v, hb_sh.at[pl.ds(s * RPT, RPT)])
    plsc.subcore_barrier()

    def body(j, carry):
        pltpu.async_copy(ones_v, ha_sh.at[src_v.at[j]], sem, add=True)
        pltpu.async_copy(ones_v, hb_sh.at[dst_v.at[j]], sem, add=True)
        return carry

    lax.fori_loop(0, NCHUNK_H, body, 0)

    def drain(j, carry):
        pltpu.make_async_copy(ones_v, ha_sh.at[src_v.at[j]], sem).wait()
        pltpu.make_async_copy(ones_v, hb_sh.at[dst_v.at[j]], sem).wait()
        return carry

    lax.fori_loop(0, NCHUNK_H, drain, 0)
    plsc.subcore_barrier()

    pltpu.sync_copy(ha_sh.at[pl.ds(s * RPT, RPT)],
                    out_hbm.at[c, 0, pl.ds(s * RPT, RPT)])
    pltpu.sync_copy(hb_sh.at[pl.ds(s * RPT, RPT)],
                    out_hbm.at[c, 1, pl.ds(s * RPT, RPT)])


def _agg_body(hs_hbm, srcr_hbm, dstr_hbm, zeros_hbm, out_hbm, src_v, dst_v,
              *rest):
    rows = rest[:NB]
    acc_sh = rest[NB]
    gsem = rest[NB + 1:2 * NB + 1]
    ssem = rest[2 * NB + 1:]
    c = lax.axis_index("c")
    s = lax.axis_index("s")

    pltpu.sync_copy(srcr_hbm.at[c, s], src_v)
    pltpu.sync_copy(dstr_hbm.at[c, s], dst_v)
    for b in range(NB):
        pltpu.async_copy(hs_hbm.at[src_v.at[b]], rows[b], gsem[b])

    pltpu.sync_copy(zeros_hbm.at[pl.ds(s * RPT, RPT)],
                    acc_sh.at[pl.ds(s * RPT, RPT)])
    plsc.subcore_barrier()

    def round_body(r, carry):
        for b in range(NB):
            j = r * NB + b
            pltpu.make_async_copy(hs_hbm.at[src_v.at[j]], rows[b],
                                  gsem[b]).wait()
            pltpu.async_copy(rows[b], acc_sh.at[dst_v.at[j]], ssem[b],
                             add=True)

        for b in range(NB):
            j = r * NB + b
            jn = j + NB
            pltpu.make_async_copy(rows[b], acc_sh.at[dst_v.at[j]],
                                  ssem[b]).wait()

            @pl.when(jn < NCHUNK)
            def _():
                pltpu.async_copy(hs_hbm.at[src_v.at[jn]], rows[b], gsem[b])

        return carry

    lax.fori_loop(0, ROUNDS, round_body, 0)
    plsc.subcore_barrier()

    pltpu.sync_copy(acc_sh.at[pl.ds(s * RPT, RPT)],
                    out_hbm.at[c, pl.ds(s * RPT, RPT)])


@functools.cache
def _sc_kernels():
    mesh = plsc.VectorSubcoreMesh(core_axis_name="c", subcore_axis_name="s",
                                  num_cores=NC, num_subcores=NS)
    params = pltpu.CompilerParams(use_tc_tiling_on_sc=False)
    hist = pl.kernel(
        _hist_body,
        out_type=jax.ShapeDtypeStruct((NC, 2, NP, HL), jnp.float32),
        mesh=mesh,
        compiler_params=params,
        scratch_types=[
            pltpu.VMEM((NCHUNK_H, KH), jnp.int32),
            pltpu.VMEM((NCHUNK_H, KH), jnp.int32),
            pltpu.VMEM((KH, HL), jnp.float32),
            pltpu.VMEM((RPT, HL), jnp.float32),
            pltpu.VMEM_SHARED((NP, HL), jnp.float32),
            pltpu.VMEM_SHARED((NP, HL), jnp.float32),
            pltpu.SemaphoreType.DMA,
        ],
    )
    agg = pl.kernel(
        _agg_body,
        out_type=jax.ShapeDtypeStruct((NC, NP, D_H), jnp.float32),
        mesh=mesh,
        compiler_params=params,
        scratch_types=[
            pltpu.VMEM((NCHUNK, K), jnp.int32),
            pltpu.VMEM((NCHUNK, K), jnp.int32),
        ] + [pltpu.VMEM((K, D_H), jnp.float32) for _ in range(NB)] + [
            pltpu.VMEM_SHARED((NP, D_H), jnp.float32),
        ] + [pltpu.SemaphoreType.DMA for _ in range(2 * NB)],
    )
    return hist, agg


_BR = 2504
_GRID = NP // _BR


def _norms(degs):
    deg_out = jnp.sum(degs[:, 0, :, 0], axis=0)
    deg_in = jnp.sum(degs[:, 1, :, 0], axis=0)
    ns = lax.rsqrt(jnp.maximum(deg_out, 1.0))
    nd = lax.rsqrt(jnp.maximum(deg_in, 1.0))
    return ns, nd


def _mm1u_body(x_ref, w_ref, out_ref):
    out_ref[...] = jnp.dot(x_ref[...], w_ref[...],
                           preferred_element_type=jnp.float32)


def _mm1u(x, w):
    return pl.pallas_call(
        _mm1u_body,
        grid=(_GRID,),
        in_specs=[
            pl.BlockSpec((_BR, D_IN), lambda i: (i, 0)),
            pl.BlockSpec((D_IN, D_H), lambda i: (0, 0)),
        ],
        out_specs=pl.BlockSpec((_BR, D_H), lambda i: (i, 0)),
        out_shape=jax.ShapeDtypeStruct((NP, D_H), jnp.float32),
    )(x, w)


def _scale_body(h_ref, degs_ref, out_ref, norms_ref):
    ns, nd = _norms(degs_ref[...])
    out_ref[...] = h_ref[...] * ns[:, None]
    norms_ref[...] = jnp.concatenate(
        [ns[:, None], nd[:, None], jnp.zeros((ns.shape[0], 6), jnp.float32)],
        axis=1)


def _scale(h, degs):
    return pl.pallas_call(
        _scale_body,
        grid=(_GRID,),
        in_specs=[
            pl.BlockSpec((_BR, D_H), lambda i: (i, 0)),
            pl.BlockSpec((NC, 2, _BR, HL), lambda i: (0, 0, i, 0)),
        ],
        out_specs=[
            pl.BlockSpec((_BR, D_H), lambda i: (i, 0)),
            pl.BlockSpec((_BR, 8), lambda i: (i, 0)),
        ],
        out_shape=[
            jax.ShapeDtypeStruct((NP, D_H), jnp.float32),
            jax.ShapeDtypeStruct((NP, 8), jnp.float32),
        ],
    )(h, degs)


def _mid_body(agg_ref, norms_ref, b_ref, w_ref, out_ref):
    ns = norms_ref[:, 0]
    nd = norms_ref[:, 1]
    a = agg_ref[0] + agg_ref[1]
    h = jnp.maximum(a * nd[:, None] + b_ref[...], 0.0)
    out_ref[...] = jnp.dot(h * ns[:, None], w_ref[...],
                           preferred_element_type=jnp.float32)


def _mid(agg, norms, b, w):
    return pl.pallas_call(
        _mid_body,
        grid=(_GRID,),
        in_specs=[
            pl.BlockSpec((NC, _BR, D_H), lambda i: (0, i, 0)),
            pl.BlockSpec((_BR, 8), lambda i: (i, 0)),
            pl.BlockSpec((1, D_H), lambda i: (0, 0)),
            pl.BlockSpec((D_H, D_H), lambda i: (0, 0)),
        ],
        out_specs=pl.BlockSpec((_BR, D_H), lambda i: (i, 0)),
        out_shape=jax.ShapeDtypeStruct((NP, D_H), jnp.float32),
    )(agg, norms, b, w)


def _fin_body(agg_ref, norms_ref, b_ref, wc_ref, bc_ref, out_ref):
    nd = norms_ref[:, 1]
    a = agg_ref[0] + agg_ref[1]
    h = jnp.maximum(a * nd[:, None] + b_ref[...], 0.0)
    out_ref[...] = jnp.dot(h, wc_ref[...],
                           preferred_element_type=jnp.float32) + bc_ref[...]


def _fin(agg, norms, b, wc, bc):
    return pl.pallas_call(
        _fin_body,
        grid=(_GRID,),
        in_specs=[
            pl.BlockSpec((NC, _BR, D_H), lambda i: (0, i, 0)),
            pl.BlockSpec((_BR, 8), lambda i: (i, 0)),
            pl.BlockSpec((1, D_H), lambda i: (0, 0)),
            pl.BlockSpec((D_H, D_OUT), lambda i: (0, 0)),
            pl.BlockSpec((1, D_OUT), lambda i: (0, 0)),
        ],
        out_specs=pl.BlockSpec((_BR, D_OUT), lambda i: (i, 0)),
        out_shape=jax.ShapeDtypeStruct((NP, D_OUT), jnp.float32),
    )(agg, norms, b, wc, bc)


def kernel(x, edge_index, W1, b1, W2, b2, Wc, bc):
    src = edge_index[0].astype(jnp.int32)
    dst = edge_index[1].astype(jnp.int32)
    srcr = src.reshape(NC, NS, NCHUNK, K)
    dstr = dst.reshape(NC, NS, NCHUNK, K)
    srcr_h = src.reshape(NC, NS, NCHUNK_H, KH)
    dstr_h = dst.reshape(NC, NS, NCHUNK_H, KH)
    xp = jnp.pad(x, ((0, NP - N), (0, 0)))
    zeros = jnp.zeros((NP, D_H), jnp.float32)

    hist_kernel, agg_kernel = _sc_kernels()
    degs = hist_kernel(srcr_h, dstr_h)
    h1u = _mm1u(xp, W1)
    hs1, norms = _scale(h1u, degs)
    agg1 = agg_kernel(hs1, srcr, dstr, zeros)
    hs2 = _mid(agg1, norms, b1.reshape(1, D_H), W2)
    agg2 = agg_kernel(hs2, srcr, dstr, zeros)
    return _fin(agg2, norms, b2.reshape(1, D_H), Wc, bc.reshape(1, D_OUT))[:N]


# speedup vs baseline: 1.0163x; 1.0163x over previous
import functools

import jax
import jax.numpy as jnp
from jax import lax
from jax.experimental import pallas as pl
from jax.experimental.pallas import tpu as pltpu
from jax.experimental.pallas import tpu_sc as plsc

N = 10000
E = 320000
D_IN = 128
D_H = 128
D_OUT = 64

NC = 2
NS = 16
NT = NC * NS
K = 40
EPT = E // NT
NCHUNK = EPT // K
NP = 10016
RPT = NP // NS
HL = 16
NB = 6
ROUNDS = -(-NCHUNK // NB)
KH = 80
NCHUNK_H = EPT // KH


def _hist_body(srcr_hbm, dstr_hbm, out_hbm, src_v, dst_v, ones_v, zero_v,
               ha_sh, hb_sh, sem):
    c = lax.axis_index("c")
    s = lax.axis_index("s")

    pltpu.sync_copy(srcr_hbm.at[c, s], src_v)
    pltpu.sync_copy(dstr_hbm.at[c, s], dst_v)

    def fill_ones(i, carry):
        ones_v[i, :] = jnp.ones((HL,), jnp.float32)
        return carry

    lax.fori_loop(0, KH, fill_ones, 0)

    def fill_zero(i, carry):
        zero_v[i, :] = jnp.zeros((HL,), jnp.float32)
        return carry

    lax.fori_loop(0, RPT, fill_zero, 0)

    pltpu.sync_copy(zero_v, ha_sh.at[pl.ds(s * RPT, RPT)])
    pltpu.sync_copy(zero_v, hb_sh.at[pl.ds(s * RPT, RPT)])
    plsc.subcore_barrier()

    def body(j, carry):
        pltpu.async_copy(ones_v, ha_sh.at[src_v.at[j]], sem, add=True)
        pltpu.async_copy(ones_v, hb_sh.at[dst_v.at[j]], sem, add=True)
        return carry

    lax.fori_loop(0, NCHUNK_H, body, 0)

    def drain(j, carry):
        pltpu.make_async_copy(ones_v, ha_sh.at[src_v.at[j]], sem).wait()
        pltpu.make_async_copy(ones_v, hb_sh.at[dst_v.at[j]], sem).wait()
        return carry

    lax.fori_loop(0, NCHUNK_H, drain, 0)
    plsc.subcore_barrier()

    pltpu.sync_copy(ha_sh.at[pl.ds(s * RPT, RPT)],
                    out_hbm.at[c, 0, pl.ds(s * RPT, RPT)])
    pltpu.sync_copy(hb_sh.at[pl.ds(s * RPT, RPT)],
                    out_hbm.at[c, 1, pl.ds(s * RPT, RPT)])


def _agg_body(hs_hbm, srcr_hbm, dstr_hbm, zeros_hbm, out_hbm, src_v, dst_v,
              *rest):
    rows = rest[:NB]
    acc_sh = rest[NB]
    gsem = rest[NB + 1:2 * NB + 1]
    ssem = rest[2 * NB + 1:]
    c = lax.axis_index("c")
    s = lax.axis_index("s")

    pltpu.sync_copy(srcr_hbm.at[c, s], src_v)
    pltpu.sync_copy(dstr_hbm.at[c, s], dst_v)
    for b in range(NB):
        pltpu.async_copy(hs_hbm.at[src_v.at[b]], rows[b], gsem[b])

    pltpu.sync_copy(zeros_hbm.at[pl.ds(s * RPT, RPT)],
                    acc_sh.at[pl.ds(s * RPT, RPT)])
    plsc.subcore_barrier()

    def round_body(r, carry):
        for b in range(NB):
            j = r * NB + b

            @pl.when(j < NCHUNK)
            def _():
                pltpu.make_async_copy(hs_hbm.at[src_v.at[j]], rows[b],
                                      gsem[b]).wait()
                pltpu.async_copy(rows[b], acc_sh.at[dst_v.at[j]], ssem[b],
                                 add=True)

        for b in range(NB):
            j = r * NB + b
            jn = j + NB

            @pl.when(j < NCHUNK)
            def _():
                pltpu.make_async_copy(rows[b], acc_sh.at[dst_v.at[j]],
                                      ssem[b]).wait()

            @pl.when(jn < NCHUNK)
            def _():
                pltpu.async_copy(hs_hbm.at[src_v.at[jn]], rows[b], gsem[b])

        return carry

    lax.fori_loop(0, ROUNDS, round_body, 0)
    plsc.subcore_barrier()

    pltpu.sync_copy(acc_sh.at[pl.ds(s * RPT, RPT)],
                    out_hbm.at[c, pl.ds(s * RPT, RPT)])


@functools.cache
def _sc_kernels():
    mesh = plsc.VectorSubcoreMesh(core_axis_name="c", subcore_axis_name="s",
                                  num_cores=NC, num_subcores=NS)
    params = pltpu.CompilerParams(use_tc_tiling_on_sc=False)
    hist = pl.kernel(
        _hist_body,
        out_type=jax.ShapeDtypeStruct((NC, 2, NP, HL), jnp.float32),
        mesh=mesh,
        compiler_params=params,
        scratch_types=[
            pltpu.VMEM((NCHUNK_H, KH), jnp.int32),
            pltpu.VMEM((NCHUNK_H, KH), jnp.int32),
            pltpu.VMEM((KH, HL), jnp.float32),
            pltpu.VMEM((RPT, HL), jnp.float32),
            pltpu.VMEM_SHARED((NP, HL), jnp.float32),
            pltpu.VMEM_SHARED((NP, HL), jnp.float32),
            pltpu.SemaphoreType.DMA,
        ],
    )
    agg = pl.kernel(
        _agg_body,
        out_type=jax.ShapeDtypeStruct((NC, NP, D_H), jnp.float32),
        mesh=mesh,
        compiler_params=params,
        scratch_types=[
            pltpu.VMEM((NCHUNK, K), jnp.int32),
            pltpu.VMEM((NCHUNK, K), jnp.int32),
        ] + [pltpu.VMEM((K, D_H), jnp.float32) for _ in range(NB)] + [
            pltpu.VMEM_SHARED((NP, D_H), jnp.float32),
        ] + [pltpu.SemaphoreType.DMA for _ in range(2 * NB)],
    )
    return hist, agg


_BR = 2504
_GRID = NP // _BR


def _norms(degs):
    deg_out = jnp.sum(degs[:, 0, :, 0], axis=0)
    deg_in = jnp.sum(degs[:, 1, :, 0], axis=0)
    ns = lax.rsqrt(jnp.maximum(deg_out, 1.0))
    nd = lax.rsqrt(jnp.maximum(deg_in, 1.0))
    return ns, nd


def _mm1u_body(x_ref, w_ref, out_ref):
    out_ref[...] = jnp.dot(x_ref[...], w_ref[...],
                           preferred_element_type=jnp.float32)


def _mm1u(x, w):
    return pl.pallas_call(
        _mm1u_body,
        grid=(_GRID,),
        in_specs=[
            pl.BlockSpec((_BR, D_IN), lambda i: (i, 0)),
            pl.BlockSpec((D_IN, D_H), lambda i: (0, 0)),
        ],
        out_specs=pl.BlockSpec((_BR, D_H), lambda i: (i, 0)),
        out_shape=jax.ShapeDtypeStruct((NP, D_H), jnp.float32),
    )(x, w)


def _scale_body(h_ref, degs_ref, out_ref, norms_ref):
    ns, nd = _norms(degs_ref[...])
    out_ref[...] = h_ref[...] * ns[:, None]
    norms_ref[...] = jnp.concatenate(
        [ns[:, None], nd[:, None], jnp.zeros((ns.shape[0], 6), jnp.float32)],
        axis=1)


def _scale(h, degs):
    return pl.pallas_call(
        _scale_body,
        grid=(_GRID,),
        in_specs=[
            pl.BlockSpec((_BR, D_H), lambda i: (i, 0)),
            pl.BlockSpec((NC, 2, _BR, HL), lambda i: (0, 0, i, 0)),
        ],
        out_specs=[
            pl.BlockSpec((_BR, D_H), lambda i: (i, 0)),
            pl.BlockSpec((_BR, 8), lambda i: (i, 0)),
        ],
        out_shape=[
            jax.ShapeDtypeStruct((NP, D_H), jnp.float32),
            jax.ShapeDtypeStruct((NP, 8), jnp.float32),
        ],
    )(h, degs)


def _mid_body(agg_ref, norms_ref, b_ref, w_ref, out_ref):
    ns = norms_ref[:, 0]
    nd = norms_ref[:, 1]
    a = agg_ref[0] + agg_ref[1]
    h = jnp.maximum(a * nd[:, None] + b_ref[...], 0.0)
    out_ref[...] = jnp.dot(h * ns[:, None], w_ref[...],
                           preferred_element_type=jnp.float32)


def _mid(agg, norms, b, w):
    return pl.pallas_call(
        _mid_body,
        grid=(_GRID,),
        in_specs=[
            pl.BlockSpec((NC, _BR, D_H), lambda i: (0, i, 0)),
            pl.BlockSpec((_BR, 8), lambda i: (i, 0)),
            pl.BlockSpec((1, D_H), lambda i: (0, 0)),
            pl.BlockSpec((D_H, D_H), lambda i: (0, 0)),
        ],
        out_specs=pl.BlockSpec((_BR, D_H), lambda i: (i, 0)),
        out_shape=jax.ShapeDtypeStruct((NP, D_H), jnp.float32),
    )(agg, norms, b, w)


def _fin_body(agg_ref, norms_ref, b_ref, wc_ref, bc_ref, out_ref):
    nd = norms_ref[:, 1]
    a = agg_ref[0] + agg_ref[1]
    h = jnp.maximum(a * nd[:, None] + b_ref[...], 0.0)
    out_ref[...] = jnp.dot(h, wc_ref[...],
                           preferred_element_type=jnp.float32) + bc_ref[...]


def _fin(agg, norms, b, wc, bc):
    return pl.pallas_call(
        _fin_body,
        grid=(_GRID,),
        in_specs=[
            pl.BlockSpec((NC, _BR, D_H), lambda i: (0, i, 0)),
            pl.BlockSpec((_BR, 8), lambda i: (i, 0)),
            pl.BlockSpec((1, D_H), lambda i: (0, 0)),
            pl.BlockSpec((D_H, D_OUT), lambda i: (0, 0)),
            pl.BlockSpec((1, D_OUT), lambda i: (0, 0)),
        ],
        out_specs=pl.BlockSpec((_BR, D_OUT), lambda i: (i, 0)),
        out_shape=jax.ShapeDtypeStruct((NP, D_OUT), jnp.float32),
    )(agg, norms, b, wc, bc)


def kernel(x, edge_index, W1, b1, W2, b2, Wc, bc):
    src = edge_index[0].astype(jnp.int32)
    dst = edge_index[1].astype(jnp.int32)
    srcr = src.reshape(NC, NS, NCHUNK, K)
    dstr = dst.reshape(NC, NS, NCHUNK, K)
    srcr_h = src.reshape(NC, NS, NCHUNK_H, KH)
    dstr_h = dst.reshape(NC, NS, NCHUNK_H, KH)
    xp = jnp.pad(x, ((0, NP - N), (0, 0)))
    zeros = jnp.zeros((NP, D_H), jnp.float32)

    hist_kernel, agg_kernel = _sc_kernels()
    degs = hist_kernel(srcr_h, dstr_h)
    h1u = _mm1u(xp, W1)
    hs1, norms = _scale(h1u, degs)
    agg1 = agg_kernel(hs1, srcr, dstr, zeros)
    hs2 = _mid(agg1, norms, b1.reshape(1, D_H), W2)
    agg2 = agg_kernel(hs2, srcr, dstr, zeros)
    return _fin(agg2, norms, b2.reshape(1, D_H), Wc, bc.reshape(1, D_OUT))[:N]
